# baseline (speedup 1.0000x reference)
import jax
import jax.numpy as jnp
from jax.experimental import pallas as pl
from jax.experimental.pallas import tpu as pltpu

_B, _T, _C = 1, 512, 256
_W = 512
_ER, _ES, _K = 16, 2, 2
_EPS = 1.1920929e-07


def _rms(x, g):
    return x * jax.lax.rsqrt(jnp.mean(x * x, axis=-1, keepdims=True) + _EPS) * g


def _gelu(x):
    return 0.5 * x * (1.0 + jax.lax.erf(x * 0.7071067811865476))


def _dense_body(u_ref, cent_ref, sg_ref, rg_ref,
                sW1_ref, sb1_ref, sW2_ref, sb2_ref,
                rW1_ref, rb1_ref, rW2_ref, rb2_ref,
                out_ref, g_scr, xnb_scr):
    e = pl.program_id(0)
    u = u_ref[...]
    ids = jax.lax.broadcasted_iota(jnp.int32, (_T, _ER), 1)
    bf = jnp.bfloat16

    @pl.when(e == 0)
    def _init():
        s = jax.nn.sigmoid(
            jnp.dot(u, cent_ref[...], preferred_element_type=jnp.float32))
        denom = jnp.sum(s, axis=1, keepdims=True)
        m1 = jnp.max(s, axis=1, keepdims=True)
        i1 = jnp.min(jnp.where(s == m1, ids, _ER), axis=1, keepdims=True)
        s2 = jnp.where(ids == i1, -jnp.inf, s)
        m2 = jnp.max(s2, axis=1, keepdims=True)
        i2 = jnp.min(jnp.where(s2 == m2, ids, _ER), axis=1, keepdims=True)
        gfull = (jnp.where(ids == i1, m1 / denom, 0.0)
                 + jnp.where(ids == i2, m2 / denom, 0.0))
        g_scr[...] = gfull
        xnb_scr[...] = _rms(u, rg_ref[0, :]).astype(bf)
        out_ref[...] = u

    @pl.when(e < _ES)
    def _shared():
        xn = _rms(u, sg_ref[0, :])
        h = _gelu((jnp.dot(xn, sW1_ref[0],
                           precision=jax.lax.Precision.DEFAULT,
                           preferred_element_type=jnp.float32)
                   + sb1_ref[0]).astype(bf))
        out_ref[...] += (jnp.dot(h, sW2_ref[0].astype(bf),
                                 preferred_element_type=jnp.float32)
                         + sb2_ref[0])

    xnb = xnb_scr[...]
    acc = jnp.zeros((_T, _C), jnp.float32)
    for sub in range(4):
        ee = e * 4 + sub
        h = _gelu((jnp.dot(xnb, rW1_ref[sub],
                           precision=jax.lax.Precision.DEFAULT,
                           preferred_element_type=jnp.float32)
                   + rb1_ref[sub]).astype(bf))
        y = jnp.dot(h, rW2_ref[sub],
                    precision=jax.lax.Precision.DEFAULT,
                    preferred_element_type=jnp.float32) + rb2_ref[sub]
        gcol = jnp.sum(jnp.where(ids == ee, g_scr[...], 0.0), axis=1,
                       keepdims=True)
        acc = acc + gcol * y
    out_ref[...] += acc


def kernel(u, shared_W1, shared_b1, shared_W2, shared_b2, shared_g,
           routed_W1, routed_b1, routed_W2, routed_b2, routed_g, centroids):
    u2 = u.reshape(_T, _C)
    out = pl.pallas_call(
        _dense_body,
        grid=(_ER // 4,),
        in_specs=[
            pl.BlockSpec((_T, _C), lambda e: (0, 0)),
            pl.BlockSpec((_C, _ER), lambda e: (0, 0)),
            pl.BlockSpec((1, _C), lambda e: (0, 0)),
            pl.BlockSpec((1, _C), lambda e: (0, 0)),
            pl.BlockSpec((1, _C, _W), lambda e: (jnp.minimum(e, _ES - 1), 0, 0)),
            pl.BlockSpec((1, 1, _W), lambda e: (jnp.minimum(e, _ES - 1), 0, 0)),
            pl.BlockSpec((1, _W, _C), lambda e: (jnp.minimum(e, _ES - 1), 0, 0)),
            pl.BlockSpec((1, 1, _C), lambda e: (jnp.minimum(e, _ES - 1), 0, 0)),
            pl.BlockSpec((4, _C, _W), lambda e: (e, 0, 0)),
            pl.BlockSpec((4, 1, _W), lambda e: (e, 0, 0)),
            pl.BlockSpec((4, _W, _C), lambda e: (e, 0, 0)),
            pl.BlockSpec((4, 1, _C), lambda e: (e, 0, 0)),
        ],
        out_specs=pl.BlockSpec((_T, _C), lambda e: (0, 0)),
        out_shape=jax.ShapeDtypeStruct((_T, _C), jnp.float32),
        scratch_shapes=[pltpu.VMEM((_T, _ER), jnp.float32),
                        pltpu.VMEM((_T, _C), jnp.bfloat16)],
        compiler_params=pltpu.CompilerParams(
            dimension_semantics=("arbitrary",),
        ),
    )(
        u2, centroids,
        shared_g.reshape(1, _C), routed_g.reshape(1, _C),
        shared_W1, shared_b1.reshape(_ES, 1, _W),
        shared_W2, shared_b2.reshape(_ES, 1, _C),
        routed_W1, routed_b1.reshape(_ER, 1, _W),
        routed_W2, routed_b2.reshape(_ER, 1, _C),
    )
    return out.reshape(_B, _T, _C)


# speedup vs baseline: 1.0034x; 1.0034x over previous
import jax
import jax.numpy as jnp
from jax.experimental import pallas as pl
from jax.experimental.pallas import tpu as pltpu

_B, _T, _C = 1, 512, 256
_W = 512
_ER, _ES, _K = 16, 2, 2
_EPS = 1.1920929e-07


def _rms(x, g):
    return x * jax.lax.rsqrt(jnp.mean(x * x, axis=-1, keepdims=True) + _EPS) * g


def _gelu(x):
    return 0.5 * x * (1.0 + jax.lax.erf(x * 0.7071067811865476))


def _dense_body(u_ref, cent_ref, sg_ref, rg_ref,
                sW1_ref, sb1_ref, sW2_ref, sb2_ref,
                rW1_ref, rb1_ref, rW2_ref, rb2_ref,
                out_ref, g_scr, xnb_scr):
    e = pl.program_id(0)
    u = u_ref[...]
    ids = jax.lax.broadcasted_iota(jnp.int32, (_T, _ER), 1)
    bf = jnp.bfloat16

    @pl.when(e == 0)
    def _init():
        s = jax.nn.sigmoid(
            jnp.dot(u, cent_ref[...], preferred_element_type=jnp.float32))
        denom = jnp.sum(s, axis=1, keepdims=True)
        m1 = jnp.max(s, axis=1, keepdims=True)
        i1 = jnp.min(jnp.where(s == m1, ids, _ER), axis=1, keepdims=True)
        s2 = jnp.where(ids == i1, -jnp.inf, s)
        m2 = jnp.max(s2, axis=1, keepdims=True)
        i2 = jnp.min(jnp.where(s2 == m2, ids, _ER), axis=1, keepdims=True)
        gfull = (jnp.where(ids == i1, m1 / denom, 0.0)
                 + jnp.where(ids == i2, m2 / denom, 0.0))
        g_scr[...] = gfull
        xnb_scr[...] = _rms(u, rg_ref[0, :]).astype(bf)
        out_ref[...] = u

    @pl.when(e < _ES)
    def _shared():
        xn = _rms(u, sg_ref[0, :])
        h = _gelu((jnp.dot(xn.astype(bf), sW1_ref[0].astype(bf),
                           preferred_element_type=jnp.float32)
                   + sb1_ref[0]).astype(bf))
        out_ref[...] += (jnp.dot(h, sW2_ref[0].astype(bf),
                                 preferred_element_type=jnp.float32)
                         + sb2_ref[0])

    xnb = xnb_scr[...]
    acc = jnp.zeros((_T, _C), jnp.float32)
    for sub in range(4):
        ee = e * 4 + sub
        h = _gelu((jnp.dot(xnb, rW1_ref[sub].astype(bf),
                           preferred_element_type=jnp.float32)
                   + rb1_ref[sub]).astype(bf))
        y = jnp.dot(h, rW2_ref[sub].astype(bf),
                    preferred_element_type=jnp.float32) + rb2_ref[sub]
        gcol = jnp.sum(jnp.where(ids == ee, g_scr[...], 0.0), axis=1,
                       keepdims=True)
        acc = acc + gcol * y
    out_ref[...] += acc


def kernel(u, shared_W1, shared_b1, shared_W2, shared_b2, shared_g,
           routed_W1, routed_b1, routed_W2, routed_b2, routed_g, centroids):
    u2 = u.reshape(_T, _C)
    out = pl.pallas_call(
        _dense_body,
        grid=(_ER // 4,),
        in_specs=[
            pl.BlockSpec((_T, _C), lambda e: (0, 0)),
            pl.BlockSpec((_C, _ER), lambda e: (0, 0)),
            pl.BlockSpec((1, _C), lambda e: (0, 0)),
            pl.BlockSpec((1, _C), lambda e: (0, 0)),
            pl.BlockSpec((1, _C, _W), lambda e: (jnp.minimum(e, _ES - 1), 0, 0)),
            pl.BlockSpec((1, 1, _W), lambda e: (jnp.minimum(e, _ES - 1), 0, 0)),
            pl.BlockSpec((1, _W, _C), lambda e: (jnp.minimum(e, _ES - 1), 0, 0)),
            pl.BlockSpec((1, 1, _C), lambda e: (jnp.minimum(e, _ES - 1), 0, 0)),
            pl.BlockSpec((4, _C, _W), lambda e: (e, 0, 0)),
            pl.BlockSpec((4, 1, _W), lambda e: (e, 0, 0)),
            pl.BlockSpec((4, _W, _C), lambda e: (e, 0, 0)),
            pl.BlockSpec((4, 1, _C), lambda e: (e, 0, 0)),
        ],
        out_specs=pl.BlockSpec((_T, _C), lambda e: (0, 0)),
        out_shape=jax.ShapeDtypeStruct((_T, _C), jnp.float32),
        scratch_shapes=[pltpu.VMEM((_T, _ER), jnp.float32),
                        pltpu.VMEM((_T, _C), jnp.bfloat16)],
        compiler_params=pltpu.CompilerParams(
            dimension_semantics=("arbitrary",),
        ),
    )(
        u2, centroids,
        shared_g.reshape(1, _C), routed_g.reshape(1, _C),
        shared_W1, shared_b1.reshape(_ES, 1, _W),
        shared_W2, shared_b2.reshape(_ES, 1, _C),
        routed_W1, routed_b1.reshape(_ER, 1, _W),
        routed_W2, routed_b2.reshape(_ER, 1, _C),
    )
    return out.reshape(_B, _T, _C)


# speedup vs baseline: 1.0379x; 1.0344x over previous
import jax
import jax.numpy as jnp
from jax.experimental import pallas as pl
from jax.experimental.pallas import tpu as pltpu

_B, _T, _C = 1, 512, 256
_W = 512
_ER, _ES, _K = 16, 2, 2
_EPS = 1.1920929e-07


def _rms(x, g):
    return x * jax.lax.rsqrt(jnp.mean(x * x, axis=-1, keepdims=True) + _EPS) * g


def _gelu(x):
    return 0.5 * x * (1.0 + jax.lax.erf(x * 0.7071067811865476))


def _dense_body(u_ref, cent_ref, sg_ref, rg_ref,
                sW1_ref, sb1_ref, sW2_ref, sb2_ref,
                rW1_ref, rb1_ref, rW2_ref, rb2_ref,
                out_ref, g_scr, xnb_scr, sW1_scr, sW2_scr, sem1, sem2):
    e = pl.program_id(0)
    u = u_ref[...]
    ids = jax.lax.broadcasted_iota(jnp.int32, (_T, _ER), 1)
    bf = jnp.bfloat16

    @pl.when(e == 0)
    def _init():
        s = jax.nn.sigmoid(
            jnp.dot(u, cent_ref[...], preferred_element_type=jnp.float32))
        denom = jnp.sum(s, axis=1, keepdims=True)
        m1 = jnp.max(s, axis=1, keepdims=True)
        i1 = jnp.min(jnp.where(s == m1, ids, _ER), axis=1, keepdims=True)
        s2 = jnp.where(ids == i1, -jnp.inf, s)
        m2 = jnp.max(s2, axis=1, keepdims=True)
        i2 = jnp.min(jnp.where(s2 == m2, ids, _ER), axis=1, keepdims=True)
        gfull = (jnp.where(ids == i1, m1 / denom, 0.0)
                 + jnp.where(ids == i2, m2 / denom, 0.0))
        g_scr[...] = gfull
        xnb_scr[...] = _rms(u, rg_ref[0, :]).astype(bf)
        out_ref[...] = u

    cp1 = pltpu.make_async_copy(sW1_ref, sW1_scr, sem1)
    cp2 = pltpu.make_async_copy(sW2_ref, sW2_scr, sem2)

    @pl.when(e == 0)
    def _start_shared_fetch():
        cp1.start()
        cp2.start()

    @pl.when(e == _ES)
    def _wait_shared_fetch():
        cp1.wait()
        cp2.wait()

    @pl.when(jnp.logical_and(e >= _ES, e < 2 * _ES))
    def _shared():
        se = e - _ES
        xn = _rms(u, sg_ref[0, :])
        h = _gelu((jnp.dot(xn.astype(bf), sW1_scr[se].astype(bf),
                           preferred_element_type=jnp.float32)
                   + sb1_ref[0]).astype(bf))
        out_ref[...] += (jnp.dot(h, sW2_scr[se].astype(bf),
                                 preferred_element_type=jnp.float32)
                         + sb2_ref[0])

    xnb = xnb_scr[...]
    acc = jnp.zeros((_T, _C), jnp.float32)
    for sub in range(4):
        ee = e * 4 + sub
        h = _gelu((jnp.dot(xnb, rW1_ref[sub].astype(bf),
                           preferred_element_type=jnp.float32)
                   + rb1_ref[sub]).astype(bf))
        y = jnp.dot(h, rW2_ref[sub].astype(bf),
                    preferred_element_type=jnp.float32) + rb2_ref[sub]
        gcol = jnp.sum(jnp.where(ids == ee, g_scr[...], 0.0), axis=1,
                       keepdims=True)
        acc = acc + gcol * y
    out_ref[...] += acc


def kernel(u, shared_W1, shared_b1, shared_W2, shared_b2, shared_g,
           routed_W1, routed_b1, routed_W2, routed_b2, routed_g, centroids):
    u2 = u.reshape(_T, _C)
    out = pl.pallas_call(
        _dense_body,
        grid=(_ER // 4,),
        in_specs=[
            pl.BlockSpec((_T, _C), lambda e: (0, 0)),
            pl.BlockSpec((_C, _ER), lambda e: (0, 0)),
            pl.BlockSpec((1, _C), lambda e: (0, 0)),
            pl.BlockSpec((1, _C), lambda e: (0, 0)),
            pl.BlockSpec(memory_space=pltpu.MemorySpace.HBM),
            pl.BlockSpec((1, 1, _W),
                         lambda e: (jnp.clip(e - _ES, 0, _ES - 1), 0, 0)),
            pl.BlockSpec(memory_space=pltpu.MemorySpace.HBM),
            pl.BlockSpec((1, 1, _C),
                         lambda e: (jnp.clip(e - _ES, 0, _ES - 1), 0, 0)),
            pl.BlockSpec((4, _C, _W), lambda e: (e, 0, 0)),
            pl.BlockSpec((4, 1, _W), lambda e: (e, 0, 0)),
            pl.BlockSpec((4, _W, _C), lambda e: (e, 0, 0)),
            pl.BlockSpec((4, 1, _C), lambda e: (e, 0, 0)),
        ],
        out_specs=pl.BlockSpec((_T, _C), lambda e: (0, 0)),
        out_shape=jax.ShapeDtypeStruct((_T, _C), jnp.float32),
        scratch_shapes=[pltpu.VMEM((_T, _ER), jnp.float32),
                        pltpu.VMEM((_T, _C), jnp.bfloat16),
                        pltpu.VMEM((_ES, _C, _W), jnp.float32),
                        pltpu.VMEM((_ES, _W, _C), jnp.float32),
                        pltpu.SemaphoreType.DMA,
                        pltpu.SemaphoreType.DMA],
        compiler_params=pltpu.CompilerParams(
            dimension_semantics=("arbitrary",),
        ),
    )(
        u2, centroids,
        shared_g.reshape(1, _C), routed_g.reshape(1, _C),
        shared_W1, shared_b1.reshape(_ES, 1, _W),
        shared_W2, shared_b2.reshape(_ES, 1, _C),
        routed_W1, routed_b1.reshape(_ER, 1, _W),
        routed_W2, routed_b2.reshape(_ER, 1, _C),
    )
    return out.reshape(_B, _T, _C)
